# baseline (speedup 1.0000x reference)
import jax
import jax.numpy as jnp
from jax import lax
from jax.experimental import pallas as pl
from jax.experimental.pallas import tpu as pltpu
from jax.experimental.pallas import tpu_sc as plsc

B, S = 1024, 200
PQ_M, SUB, CODEBOOK, EMB = 8, 16, 256, 128
N = B * S
NC, NS = 2, 16
NW = NC * NS
TPW = N // NW
BLK = 256
NBLK = TPW // BLK
CODES_BLK = BLK * PQ_M
IDX_CHUNK = 128
A_CHUNKS = BLK // IDX_CHUNK
C_CHUNKS = CODES_BLK // IDX_CHUNK
NUM_ITEMS = 1000000


def _sc_body(ids_hbm, codesT_hbm, cent_hbm, out_hbm,
             ids_v, codes_v, pos_v, tmp_v, cent_sh, sem):
    sid = lax.axis_index("s")
    wid = sid * NC + lax.axis_index("c")
    lane = lax.iota(jnp.int32, 16)

    @pl.when(sid == 0)
    def _stage_table():
        pltpu.sync_copy(cent_hbm, cent_sh)

    def pos_body(i, c2):
        m = i // (8 * A_CHUNKS)
        tloc = ((i // 8) % A_CHUNKS) * IDX_CHUNK + (i % 8) * 16
        pos_v[i // 8, pl.ds((i % 8) * 16, 16)] = lane * PQ_M + (m + PQ_M * tloc)
        return c2
    lax.fori_loop(0, CODES_BLK // 16, pos_body, 0)
    plsc.subcore_barrier()

    def block_body(blk, carry):
        t0 = wid * TPW + blk * BLK
        pltpu.sync_copy(ids_hbm.at[pl.ds(t0, BLK)], ids_v)

        hs = [
            pltpu.async_copy(
                codesT_hbm.at[m].at[ids_v.at[pl.ds(c * IDX_CHUNK, IDX_CHUNK)]],
                codes_v.at[pl.ds((m * A_CHUNKS + c) * IDX_CHUNK, IDX_CHUNK)],
                sem)
            for m in range(PQ_M) for c in range(A_CHUNKS)
        ]
        for h in hs:
            h.wait()

        hs = [
            pltpu.async_copy(
                cent_sh.at[pl.ds((j // A_CHUNKS) * CODEBOOK, CODEBOOK)]
                       .at[codes_v.at[pl.ds(j * IDX_CHUNK, IDX_CHUNK)]],
                tmp_v.at[pl.ds(j * IDX_CHUNK, IDX_CHUNK)],
                sem)
            for j in range(C_CHUNKS)
        ]
        for h in hs:
            h.wait()

        owin = out_hbm.at[pl.ds(t0 * PQ_M, CODES_BLK)]
        hs = [
            pltpu.async_copy(
                tmp_v.at[pl.ds(j * IDX_CHUNK, IDX_CHUNK)],
                owin.at[pos_v.at[j]],
                sem)
            for j in range(C_CHUNKS)
        ]
        for h in hs:
            h.wait()
        return carry

    lax.fori_loop(0, NBLK, block_body, 0)


def kernel(input_ids, item_codes, centroids):
    ids_flat = input_ids.reshape(N)
    codes_t = item_codes.T
    cent_flat = centroids.reshape(PQ_M * CODEBOOK, SUB)
    mesh = plsc.VectorSubcoreMesh(core_axis_name="c", subcore_axis_name="s")
    f = pl.kernel(
        _sc_body,
        mesh=mesh,
        compiler_params=pltpu.CompilerParams(use_tc_tiling_on_sc=False),
        out_type=jax.ShapeDtypeStruct((N * PQ_M, SUB), jnp.float32),
        scratch_types=[
            pltpu.VMEM((BLK,), jnp.int32),
            pltpu.VMEM((CODES_BLK,), jnp.int32),
            pltpu.VMEM((C_CHUNKS, IDX_CHUNK), jnp.int32),
            pltpu.VMEM((CODES_BLK, SUB), jnp.float32),
            pltpu.VMEM_SHARED((PQ_M * CODEBOOK, SUB), jnp.float32),
            pltpu.SemaphoreType.DMA,
        ],
    )
    out = f(ids_flat, codes_t, cent_flat)
    return out.reshape(B, S, EMB)


# speedup vs baseline: 3.6866x; 3.6866x over previous
import jax
import jax.numpy as jnp
from jax import lax
from jax.experimental import pallas as pl
from jax.experimental.pallas import tpu as pltpu
from jax.experimental.pallas import tpu_sc as plsc

B, S = 1024, 200
PQ_M, SUB, CODEBOOK, EMB = 8, 16, 256, 128
N = B * S
NC, NS = 2, 16
NW = NC * NS
TPW = N // NW
BLK = 256
NBLK = TPW // BLK
CODES_BLK = BLK * PQ_M
IDX_CHUNK = 128
C_CHUNKS = CODES_BLK // IDX_CHUNK
NUM_ITEMS = 1000000
MAIN_ITEMS = 999936
MAIN_WORDS = MAIN_ITEMS * PQ_M
AUX_ROWS = NUM_ITEMS + 1 - MAIN_ITEMS
AUX_WORDS = AUX_ROWS * PQ_M


def _sc_body(ids_hbm, codes_hbm, cent_hbm, out_hbm,
             ids_v, fidx_v, auxi_v, tail_v, codes_v, out_v,
             cent_sh, sem):
    sid = lax.axis_index("s")
    wid = sid * NC + lax.axis_index("c")
    lane = lax.iota(jnp.int32, 16)
    colpat = lax.bitwise_and(lane, 7)
    rowpat = lax.shift_right_logical(lane, 3)
    mpat = colpat * CODEBOOK
    m128 = colpat * IDX_CHUNK

    @pl.when(sid == 0)
    def _stage_tables():
        pltpu.sync_copy(cent_hbm, cent_sh)
    plsc.subcore_barrier()

    def block_body(blk, carry):
        t0 = wid * TPW + blk * BLK
        pltpu.sync_copy(ids_hbm.at[pl.ds(t0, BLK)], ids_v)

        def eidx_body(i, c2):
            ids16 = ids_v[pl.ds(i * 16, 16)]
            for p in range(8):
                toks = ids16.at[rowpat + 2 * p].get(
                    mode="promise_in_bounds")
                o = i * 128 + p * 16
                tmain = jnp.minimum(toks, MAIN_ITEMS - 1)
                fidx_v[pl.ds(o, 16)] = (
                    lax.shift_right_logical(tmain, 7) * 1024 + m128
                    + lax.bitwise_and(tmain, 127))
                taux = jnp.maximum(toks - MAIN_ITEMS, 0)
                auxi_v[pl.ds(o, 16)] = (
                    taux * PQ_M + colpat + PQ_M * CODEBOOK)
                tail_v[pl.ds(o, 16)] = jnp.where(
                    toks >= MAIN_ITEMS, 1, 0)
            return c2
        lax.fori_loop(0, BLK // 16, eidx_body, 0)

        hs = [
            pltpu.async_copy(
                codes_hbm.at[fidx_v.at[pl.ds(c * IDX_CHUNK, IDX_CHUNK)]],
                codes_v.at[pl.ds(c * IDX_CHUNK, IDX_CHUNK)],
                sem)
            for c in range(C_CHUNKS)
        ]
        for h in hs:
            h.wait()

        def fidx_body(i, c2):
            g = jnp.where(tail_v[pl.ds(i * 16, 16)] > 0,
                          auxi_v[pl.ds(i * 16, 16)],
                          codes_v[pl.ds(i * 16, 16)] + mpat)
            fidx_v[pl.ds(i * 16, 16)] = g
            return c2
        lax.fori_loop(0, CODES_BLK // 16, fidx_body, 0)

        hs = [
            pltpu.async_copy(
                cent_sh.at[fidx_v.at[pl.ds(c * IDX_CHUNK, IDX_CHUNK)]],
                out_v.at[pl.ds(c * IDX_CHUNK, IDX_CHUNK)],
                sem)
            for c in range(C_CHUNKS)
        ]
        for h in hs:
            h.wait()

        pltpu.sync_copy(out_v, out_hbm.at[pl.ds(t0 * PQ_M, CODES_BLK)])
        return carry

    lax.fori_loop(0, NBLK, block_body, 0)


def kernel(input_ids, item_codes, centroids):
    ids_flat = input_ids.reshape(N)
    codes_main = (item_codes[:MAIN_ITEMS]
                  .reshape(MAIN_ITEMS // IDX_CHUNK, IDX_CHUNK, PQ_M)
                  .transpose(0, 2, 1)
                  .reshape(MAIN_WORDS))
    codes_aux = item_codes[MAIN_ITEMS:]
    tail_embs = centroids[
        jnp.arange(PQ_M, dtype=jnp.int32)[None, :], codes_aux
    ].reshape(AUX_WORDS, SUB)
    cent_flat = jnp.concatenate(
        [centroids.reshape(PQ_M * CODEBOOK, SUB), tail_embs], axis=0)
    mesh = plsc.VectorSubcoreMesh(core_axis_name="c", subcore_axis_name="s")
    f = pl.kernel(
        _sc_body,
        mesh=mesh,
        compiler_params=pltpu.CompilerParams(use_tc_tiling_on_sc=False),
        out_type=jax.ShapeDtypeStruct((N * PQ_M, SUB), jnp.float32),
        scratch_types=[
            pltpu.VMEM((BLK,), jnp.int32),
            pltpu.VMEM((CODES_BLK,), jnp.int32),
            pltpu.VMEM((CODES_BLK,), jnp.int32),
            pltpu.VMEM((CODES_BLK,), jnp.int32),
            pltpu.VMEM((CODES_BLK,), jnp.int32),
            pltpu.VMEM((CODES_BLK, SUB), jnp.float32),
            pltpu.VMEM_SHARED((PQ_M * CODEBOOK + AUX_WORDS, SUB),
                              jnp.float32),
            pltpu.SemaphoreType.DMA,
        ],
    )
    out = f(ids_flat, codes_main, cent_flat)
    return out.reshape(B, S, EMB)


# speedup vs baseline: 4.4472x; 1.2063x over previous
import jax
import jax.numpy as jnp
from jax import lax
from jax.experimental import pallas as pl
from jax.experimental.pallas import tpu as pltpu
from jax.experimental.pallas import tpu_sc as plsc

B, S = 1024, 200
PQ_M, SUB, CODEBOOK, EMB = 8, 16, 256, 128
N = B * S
NC, NS = 2, 16
NW = NC * NS
TPW = N // NW
BLK = 256
NBLK = TPW // BLK
CODES_BLK = BLK * PQ_M
IDX_CHUNK = 128
C_CHUNKS = CODES_BLK // IDX_CHUNK
NUM_ITEMS = 1000000
MAIN_ITEMS = 999936
MAIN_WORDS = MAIN_ITEMS * PQ_M
AUX_ROWS = NUM_ITEMS + 1 - MAIN_ITEMS
AUX_WORDS = AUX_ROWS * PQ_M


def _sc_body(ids_hbm, codes_hbm, cent_hbm, out_hbm,
             ids_v, fidx_v, auxi_v, tail_v, codes_v, out0_v, out1_v,
             cent_sh, semA, semC, semE0, semE1):
    sid = lax.axis_index("s")
    wid = sid * NC + lax.axis_index("c")
    lane = lax.iota(jnp.int32, 16)
    colpat = lax.bitwise_and(lane, 7)
    rowpat = lax.shift_right_logical(lane, 3)
    mpat = colpat * CODEBOOK
    m128 = colpat * IDX_CHUNK

    @pl.when(sid == 0)
    def _stage_tables():
        pltpu.sync_copy(cent_hbm, cent_sh)
    plsc.subcore_barrier()

    def do_block(blk, out_v, semE, wait_e):
        t0 = wid * TPW + blk * BLK
        pltpu.sync_copy(ids_hbm.at[pl.ds(t0, BLK)], ids_v)

        def eidx_body(i, c2):
            ids16 = ids_v[pl.ds(i * 16, 16)]
            for p in range(8):
                toks = ids16.at[rowpat + 2 * p].get(
                    mode="promise_in_bounds")
                o = i * 128 + p * 16
                tmain = jnp.minimum(toks, MAIN_ITEMS - 1)
                fidx_v[pl.ds(o, 16)] = (
                    lax.shift_right_logical(tmain, 7) * 1024 + m128
                    + lax.bitwise_and(tmain, 127))
                taux = jnp.maximum(toks - MAIN_ITEMS, 0)
                auxi_v[pl.ds(o, 16)] = (
                    taux * PQ_M + colpat + PQ_M * CODEBOOK)
                tail_v[pl.ds(o, 16)] = jnp.where(
                    toks >= MAIN_ITEMS, 1, 0)
            pltpu.async_copy(
                codes_hbm.at[fidx_v.at[pl.ds(i * IDX_CHUNK, IDX_CHUNK)]],
                codes_v.at[pl.ds(i * IDX_CHUNK, IDX_CHUNK)],
                semA)
            return c2
        lax.fori_loop(0, C_CHUNKS, eidx_body, 0)

        @pl.when(wait_e)
        def _drain_prev_out():
            pltpu.make_async_copy(
                out_v, out_hbm.at[pl.ds(t0 * PQ_M, CODES_BLK)], semE
            ).wait()

        for c in range(C_CHUNKS):
            pltpu.make_async_copy(
                codes_hbm.at[fidx_v.at[pl.ds(c * IDX_CHUNK, IDX_CHUNK)]],
                codes_v.at[pl.ds(c * IDX_CHUNK, IDX_CHUNK)],
                semA).wait()

        def fidx_body(i, c2):
            for q in range(8):
                o = i * 128 + q * 16
                g = jnp.where(tail_v[pl.ds(o, 16)] > 0,
                              auxi_v[pl.ds(o, 16)],
                              codes_v[pl.ds(o, 16)] + mpat)
                fidx_v[pl.ds(o, 16)] = g
            pltpu.async_copy(
                cent_sh.at[fidx_v.at[pl.ds(i * IDX_CHUNK, IDX_CHUNK)]],
                out_v.at[pl.ds(i * IDX_CHUNK, IDX_CHUNK)],
                semC)
            return c2
        lax.fori_loop(0, C_CHUNKS, fidx_body, 0)
        for c in range(C_CHUNKS):
            pltpu.make_async_copy(
                cent_sh.at[fidx_v.at[pl.ds(c * IDX_CHUNK, IDX_CHUNK)]],
                out_v.at[pl.ds(c * IDX_CHUNK, IDX_CHUNK)],
                semC).wait()

        pltpu.async_copy(
            out_v, out_hbm.at[pl.ds(t0 * PQ_M, CODES_BLK)], semE)

    def pair_body(k, carry):
        do_block(2 * k, out0_v, semE0, k > 0)
        do_block(2 * k + 1, out1_v, semE1, k > 0)
        return carry
    lax.fori_loop(0, NBLK // 2, pair_body, 0)
    do_block(NBLK - 1, out0_v, semE0, True)
    last0 = wid * TPW + (NBLK - 1) * BLK
    pltpu.make_async_copy(
        out0_v, out_hbm.at[pl.ds(last0 * PQ_M, CODES_BLK)], semE0).wait()
    last1 = wid * TPW + (NBLK - 2) * BLK
    pltpu.make_async_copy(
        out1_v, out_hbm.at[pl.ds(last1 * PQ_M, CODES_BLK)], semE1).wait()


def kernel(input_ids, item_codes, centroids):
    ids_flat = input_ids.reshape(N)
    codes_main = (item_codes[:MAIN_ITEMS]
                  .reshape(MAIN_ITEMS // IDX_CHUNK, IDX_CHUNK, PQ_M)
                  .transpose(0, 2, 1)
                  .reshape(MAIN_WORDS))
    codes_aux = item_codes[MAIN_ITEMS:]
    tail_embs = centroids[
        jnp.arange(PQ_M, dtype=jnp.int32)[None, :], codes_aux
    ].reshape(AUX_WORDS, SUB)
    cent_flat = jnp.concatenate(
        [centroids.reshape(PQ_M * CODEBOOK, SUB), tail_embs], axis=0)
    mesh = plsc.VectorSubcoreMesh(core_axis_name="c", subcore_axis_name="s")
    f = pl.kernel(
        _sc_body,
        mesh=mesh,
        compiler_params=pltpu.CompilerParams(use_tc_tiling_on_sc=False),
        out_type=jax.ShapeDtypeStruct((N * PQ_M, SUB), jnp.float32),
        scratch_types=[
            pltpu.VMEM((BLK,), jnp.int32),
            pltpu.VMEM((CODES_BLK,), jnp.int32),
            pltpu.VMEM((CODES_BLK,), jnp.int32),
            pltpu.VMEM((CODES_BLK,), jnp.int32),
            pltpu.VMEM((CODES_BLK,), jnp.int32),
            pltpu.VMEM((CODES_BLK, SUB), jnp.float32),
            pltpu.VMEM((CODES_BLK, SUB), jnp.float32),
            pltpu.VMEM_SHARED((PQ_M * CODEBOOK + AUX_WORDS, SUB),
                              jnp.float32),
            pltpu.SemaphoreType.DMA,
            pltpu.SemaphoreType.DMA,
            pltpu.SemaphoreType.DMA,
            pltpu.SemaphoreType.DMA,
        ],
    )
    out = f(ids_flat, codes_main, cent_flat)
    return out.reshape(B, S, EMB)


# speedup vs baseline: 4.4934x; 1.0104x over previous
import jax
import jax.numpy as jnp
from jax import lax
from jax.experimental import pallas as pl
from jax.experimental.pallas import tpu as pltpu
from jax.experimental.pallas import tpu_sc as plsc

B, S = 1024, 200
PQ_M, SUB, CODEBOOK, EMB = 8, 16, 256, 128
N = B * S
NC, NS = 2, 16
NW = NC * NS
TPW = N // NW
BLK = 256
NBLK = TPW // BLK
CODES_BLK = BLK * PQ_M
IDX_CHUNK = 128
C_CHUNKS = CODES_BLK // IDX_CHUNK
NUM_ITEMS = 1000000
MAIN_ITEMS = 999936
MAIN_WORDS = MAIN_ITEMS * PQ_M
AUX_ROWS = NUM_ITEMS + 1 - MAIN_ITEMS
AUX_WORDS = AUX_ROWS * PQ_M


def _sc_body(ids_hbm, codes_hbm, cent_hbm, out_hbm,
             ids_v, fidx_v, auxi_v, tail_v, codes_v, out0_v, out1_v,
             cent_sh, semA, semC, semE0, semE1):
    sid = lax.axis_index("s")
    wid = sid * NC + lax.axis_index("c")
    lane = lax.iota(jnp.int32, 16)
    colpat = lax.bitwise_and(lane, 7)
    rowpat = lax.shift_right_logical(lane, 3)
    mpat = colpat * CODEBOOK
    m128 = colpat * IDX_CHUNK

    @pl.when(sid == 0)
    def _stage_tables():
        pltpu.sync_copy(cent_hbm, cent_sh)
    plsc.subcore_barrier()

    def do_block(blk, out_v, semE, wait_e):
        t0 = wid * TPW + blk * BLK
        pltpu.sync_copy(ids_hbm.at[pl.ds(t0, BLK)], ids_v)

        def eidx_body(i, c2):
            ids16 = ids_v[pl.ds(i * 16, 16)]
            for p in range(8):
                toks = ids16.at[rowpat + 2 * p].get(
                    mode="promise_in_bounds")
                o = i * 128 + p * 16
                tmain = jnp.minimum(toks, MAIN_ITEMS - 1)
                fidx_v[pl.ds(o, 16)] = (
                    lax.shift_right_logical(tmain, 7) * 1024 + m128
                    + lax.bitwise_and(tmain, 127))
                taux = jnp.maximum(toks - MAIN_ITEMS, 0)
                auxi_v[pl.ds(o, 16)] = (
                    taux * PQ_M + colpat + PQ_M * CODEBOOK)
                tail_v[pl.ds(o, 16)] = jnp.where(
                    toks >= MAIN_ITEMS, 1, 0)
            pltpu.async_copy(
                codes_hbm.at[fidx_v.at[pl.ds(i * IDX_CHUNK, IDX_CHUNK)]],
                codes_v.at[pl.ds(i * IDX_CHUNK, IDX_CHUNK)],
                semA)
            return c2
        lax.fori_loop(0, C_CHUNKS, eidx_body, 0)

        @pl.when(wait_e)
        def _drain_prev_out():
            pltpu.make_async_copy(
                out_v, out_hbm.at[pl.ds(t0 * PQ_M, CODES_BLK)], semE
            ).wait()

        def fidx_body(i, c2):
            pltpu.make_async_copy(
                codes_hbm.at[fidx_v.at[pl.ds(i * IDX_CHUNK, IDX_CHUNK)]],
                codes_v.at[pl.ds(i * IDX_CHUNK, IDX_CHUNK)],
                semA).wait()
            for q in range(8):
                o = i * 128 + q * 16
                g = jnp.where(tail_v[pl.ds(o, 16)] > 0,
                              auxi_v[pl.ds(o, 16)],
                              codes_v[pl.ds(o, 16)] + mpat)
                fidx_v[pl.ds(o, 16)] = g
            pltpu.async_copy(
                cent_sh.at[fidx_v.at[pl.ds(i * IDX_CHUNK, IDX_CHUNK)]],
                out_v.at[pl.ds(i * IDX_CHUNK, IDX_CHUNK)],
                semC)
            return c2
        lax.fori_loop(0, C_CHUNKS, fidx_body, 0)
        for c in range(C_CHUNKS):
            pltpu.make_async_copy(
                cent_sh.at[fidx_v.at[pl.ds(c * IDX_CHUNK, IDX_CHUNK)]],
                out_v.at[pl.ds(c * IDX_CHUNK, IDX_CHUNK)],
                semC).wait()

        pltpu.async_copy(
            out_v, out_hbm.at[pl.ds(t0 * PQ_M, CODES_BLK)], semE)

    def pair_body(k, carry):
        do_block(2 * k, out0_v, semE0, k > 0)
        do_block(2 * k + 1, out1_v, semE1, k > 0)
        return carry
    lax.fori_loop(0, NBLK // 2, pair_body, 0)
    do_block(NBLK - 1, out0_v, semE0, True)
    last0 = wid * TPW + (NBLK - 1) * BLK
    pltpu.make_async_copy(
        out0_v, out_hbm.at[pl.ds(last0 * PQ_M, CODES_BLK)], semE0).wait()
    last1 = wid * TPW + (NBLK - 2) * BLK
    pltpu.make_async_copy(
        out1_v, out_hbm.at[pl.ds(last1 * PQ_M, CODES_BLK)], semE1).wait()


def kernel(input_ids, item_codes, centroids):
    ids_flat = input_ids.reshape(N)
    codes_main = (item_codes[:MAIN_ITEMS]
                  .reshape(MAIN_ITEMS // IDX_CHUNK, IDX_CHUNK, PQ_M)
                  .transpose(0, 2, 1)
                  .reshape(MAIN_WORDS))
    codes_aux = item_codes[MAIN_ITEMS:]
    tail_embs = centroids[
        jnp.arange(PQ_M, dtype=jnp.int32)[None, :], codes_aux
    ].reshape(AUX_WORDS, SUB)
    cent_flat = jnp.concatenate(
        [centroids.reshape(PQ_M * CODEBOOK, SUB), tail_embs], axis=0)
    mesh = plsc.VectorSubcoreMesh(core_axis_name="c", subcore_axis_name="s")
    f = pl.kernel(
        _sc_body,
        mesh=mesh,
        compiler_params=pltpu.CompilerParams(use_tc_tiling_on_sc=False),
        out_type=jax.ShapeDtypeStruct((N * PQ_M, SUB), jnp.float32),
        scratch_types=[
            pltpu.VMEM((BLK,), jnp.int32),
            pltpu.VMEM((CODES_BLK,), jnp.int32),
            pltpu.VMEM((CODES_BLK,), jnp.int32),
            pltpu.VMEM((CODES_BLK,), jnp.int32),
            pltpu.VMEM((CODES_BLK,), jnp.int32),
            pltpu.VMEM((CODES_BLK, SUB), jnp.float32),
            pltpu.VMEM((CODES_BLK, SUB), jnp.float32),
            pltpu.VMEM_SHARED((PQ_M * CODEBOOK + AUX_WORDS, SUB),
                              jnp.float32),
            pltpu.SemaphoreType.DMA,
            pltpu.SemaphoreType.DMA,
            pltpu.SemaphoreType.DMA,
            pltpu.SemaphoreType.DMA,
        ],
    )
    out = f(ids_flat, codes_main, cent_flat)
    return out.reshape(B, S, EMB)


# speedup vs baseline: 4.5053x; 1.0026x over previous
import jax
import jax.numpy as jnp
from jax import lax
from jax.experimental import pallas as pl
from jax.experimental.pallas import tpu as pltpu
from jax.experimental.pallas import tpu_sc as plsc

B, S = 1024, 200
PQ_M, SUB, CODEBOOK, EMB = 8, 16, 256, 128
N = B * S
NC, NS = 2, 16
NW = NC * NS
TPW = N // NW
BLK = 256
NBLK = TPW // BLK
CODES_BLK = BLK * PQ_M
IDX_CHUNK = 128
C_CHUNKS = CODES_BLK // IDX_CHUNK
NUM_ITEMS = 1000000
MAIN_ITEMS = 999936
MAIN_WORDS = MAIN_ITEMS * PQ_M
AUX_ROWS = NUM_ITEMS + 1 - MAIN_ITEMS
AUX_WORDS = AUX_ROWS * PQ_M


def _sc_body(ids_hbm, codes_hbm, cent_hbm, out_hbm,
             ids_v, fidx_v, auxi_v, codes_v, out0_v, out1_v,
             cent_sh, semA, semC, semE0, semE1):
    sid = lax.axis_index("s")
    wid = sid * NC + lax.axis_index("c")
    lane = lax.iota(jnp.int32, 16)
    colpat = lax.bitwise_and(lane, 7)
    rowpat = lax.shift_right_logical(lane, 3)
    mpat = colpat * CODEBOOK
    m128 = colpat * IDX_CHUNK

    @pl.when(sid == 0)
    def _stage_tables():
        pltpu.sync_copy(cent_hbm, cent_sh)
    plsc.subcore_barrier()

    def do_block(blk, out_v, semE, wait_e):
        t0 = wid * TPW + blk * BLK
        pltpu.sync_copy(ids_hbm.at[pl.ds(t0, BLK)], ids_v)

        def eidx_body(i, c2):
            ids16 = ids_v[pl.ds(i * 16, 16)]
            tmain = jnp.minimum(ids16, MAIN_ITEMS - 1)
            base16 = (lax.shift_right_logical(tmain, 7) * 1024
                      + lax.bitwise_and(tmain, 127))
            auxb16 = jnp.where(
                ids16 >= MAIN_ITEMS,
                (ids16 - MAIN_ITEMS) * PQ_M + PQ_M * CODEBOOK, 0)
            for p in range(8):
                rp = rowpat + 2 * p
                o = i * 128 + p * 16
                fidx_v[pl.ds(o, 16)] = base16.at[rp].get(
                    mode="promise_in_bounds") + m128
                auxi_v[pl.ds(o, 16)] = auxb16.at[rp].get(
                    mode="promise_in_bounds") + colpat
            pltpu.async_copy(
                codes_hbm.at[fidx_v.at[pl.ds(i * IDX_CHUNK, IDX_CHUNK)]],
                codes_v.at[pl.ds(i * IDX_CHUNK, IDX_CHUNK)],
                semA)
            return c2
        lax.fori_loop(0, C_CHUNKS, eidx_body, 0)

        @pl.when(wait_e)
        def _drain_prev_out():
            pltpu.make_async_copy(
                out_v, out_hbm.at[pl.ds(t0 * PQ_M, CODES_BLK)], semE
            ).wait()

        def fidx_body(i, c2):
            pltpu.make_async_copy(
                codes_hbm.at[fidx_v.at[pl.ds(i * IDX_CHUNK, IDX_CHUNK)]],
                codes_v.at[pl.ds(i * IDX_CHUNK, IDX_CHUNK)],
                semA).wait()
            for q in range(8):
                o = i * 128 + q * 16
                fidx_v[pl.ds(o, 16)] = jnp.maximum(
                    codes_v[pl.ds(o, 16)] + mpat, auxi_v[pl.ds(o, 16)])
            pltpu.async_copy(
                cent_sh.at[fidx_v.at[pl.ds(i * IDX_CHUNK, IDX_CHUNK)]],
                out_v.at[pl.ds(i * IDX_CHUNK, IDX_CHUNK)],
                semC)
            return c2
        lax.fori_loop(0, C_CHUNKS, fidx_body, 0)
        for c in range(C_CHUNKS):
            pltpu.make_async_copy(
                cent_sh.at[fidx_v.at[pl.ds(c * IDX_CHUNK, IDX_CHUNK)]],
                out_v.at[pl.ds(c * IDX_CHUNK, IDX_CHUNK)],
                semC).wait()

        pltpu.async_copy(
            out_v, out_hbm.at[pl.ds(t0 * PQ_M, CODES_BLK)], semE)

    def pair_body(k, carry):
        do_block(2 * k, out0_v, semE0, k > 0)
        do_block(2 * k + 1, out1_v, semE1, k > 0)
        return carry
    lax.fori_loop(0, NBLK // 2, pair_body, 0)
    do_block(NBLK - 1, out0_v, semE0, True)
    last0 = wid * TPW + (NBLK - 1) * BLK
    pltpu.make_async_copy(
        out0_v, out_hbm.at[pl.ds(last0 * PQ_M, CODES_BLK)], semE0).wait()
    last1 = wid * TPW + (NBLK - 2) * BLK
    pltpu.make_async_copy(
        out1_v, out_hbm.at[pl.ds(last1 * PQ_M, CODES_BLK)], semE1).wait()


def kernel(input_ids, item_codes, centroids):
    ids_flat = input_ids.reshape(N)
    codes_main = (item_codes[:MAIN_ITEMS]
                  .reshape(MAIN_ITEMS // IDX_CHUNK, IDX_CHUNK, PQ_M)
                  .transpose(0, 2, 1)
                  .reshape(MAIN_WORDS))
    codes_aux = item_codes[MAIN_ITEMS:]
    tail_embs = centroids[
        jnp.arange(PQ_M, dtype=jnp.int32)[None, :], codes_aux
    ].reshape(AUX_WORDS, SUB)
    cent_flat = jnp.concatenate(
        [centroids.reshape(PQ_M * CODEBOOK, SUB), tail_embs], axis=0)
    mesh = plsc.VectorSubcoreMesh(core_axis_name="c", subcore_axis_name="s")
    f = pl.kernel(
        _sc_body,
        mesh=mesh,
        compiler_params=pltpu.CompilerParams(use_tc_tiling_on_sc=False),
        out_type=jax.ShapeDtypeStruct((N * PQ_M, SUB), jnp.float32),
        scratch_types=[
            pltpu.VMEM((BLK,), jnp.int32),
            pltpu.VMEM((CODES_BLK,), jnp.int32),
            pltpu.VMEM((CODES_BLK,), jnp.int32),
            pltpu.VMEM((CODES_BLK,), jnp.int32),
            pltpu.VMEM((CODES_BLK, SUB), jnp.float32),
            pltpu.VMEM((CODES_BLK, SUB), jnp.float32),
            pltpu.VMEM_SHARED((PQ_M * CODEBOOK + AUX_WORDS, SUB),
                              jnp.float32),
            pltpu.SemaphoreType.DMA,
            pltpu.SemaphoreType.DMA,
            pltpu.SemaphoreType.DMA,
            pltpu.SemaphoreType.DMA,
        ],
    )
    out = f(ids_flat, codes_main, cent_flat)
    return out.reshape(B, S, EMB)
